# baseline (speedup 1.0000x reference)
import jax
import jax.numpy as jnp
from jax.experimental import pallas as pl
from jax.experimental.pallas import tpu as pltpu

_B, _F, _T = 128, 80, 4096
_FREQ_MASKS = 2
_TIME_MASKS = 10
_S = 8
_NCHUNK = _F // _S


def _spans(fs_ref, fl_ref):
    s0, l0 = fs_ref[0], fl_ref[0]
    s1, l1 = fs_ref[1], fl_ref[1]
    e0, e1 = s0 + l0, s1 + l1
    first = s0 <= s1
    a_s = jnp.where(first, s0, s1)
    a_e = jnp.where(first, e0, e1)
    b_s = jnp.where(first, s1, s0)
    b_e = jnp.where(first, e1, e0)
    merged = b_s <= a_e
    i0s = a_s
    i0e = jnp.where(merged, jnp.maximum(a_e, b_e), a_e)
    i1s = jnp.where(merged, _F, b_s)
    i1e = jnp.where(merged, _F, b_e)
    return ((jnp.int32(0), i0s), (i0e, i1s), (i1e, jnp.int32(_F)))


def _issue(spans, b_src, slot, x_ref, xbuf_ref, sem_ref, do_wait):
    for ss, se in spans:
        base = (ss // _S) * _S
        for k in range(_NCHUNK):
            start = base + k * _S

            @pl.when(start < se)
            def _(start=start):
                start = pl.multiple_of(start, _S)
                cp = pltpu.make_async_copy(
                    x_ref.at[b_src, pl.ds(start, _S), :],
                    xbuf_ref.at[slot, pl.ds(start, _S), :],
                    sem_ref.at[slot],
                )
                if do_wait:
                    cp.wait()
                else:
                    cp.start()


def _body(xlen_ref, fs_ref, fl_ref, ts_ref, tl_ref, x_ref, o_ref,
          xbuf_ref, tkeep_ref, sem_ref):
    b = pl.program_id(0)
    spans = _spans(fs_ref, fl_ref)

    @pl.when(b == 0)
    def _():
        _issue(spans, 0, 0, x_ref, xbuf_ref, sem_ref, do_wait=False)
        t_io = jax.lax.broadcasted_iota(jnp.int32, (1, _T), 1)
        tk = jnp.ones((1, _T), jnp.float32)
        for i in range(_TIME_MASKS):
            s = ts_ref[i]
            e = s + tl_ref[i]
            tk = jnp.where((t_io >= s) & (t_io < e), 0.0, tk)
        tkeep_ref[...] = tk

    @pl.when(b + 1 < _B)
    def _():
        _issue(spans, b + 1, (b + 1) % 2, x_ref, xbuf_ref, sem_ref,
               do_wait=False)

    _issue(spans, b, b % 2, x_ref, xbuf_ref, sem_ref, do_wait=True)

    f_io = jax.lax.broadcasted_iota(jnp.int32, (_F, 1), 0)
    fkeep = jnp.ones((_F, 1), jnp.float32)
    for i in range(_FREQ_MASKS):
        s = fs_ref[i]
        e = s + fl_ref[i]
        fkeep = jnp.where((f_io >= s) & (f_io < e), 0.0, fkeep)

    xl = xlen_ref[b]
    t_io = jax.lax.broadcasted_iota(jnp.int32, (1, _T), 1)
    tkeep = jnp.where(t_io < xl, tkeep_ref[...], 1.0)

    keep = fkeep * tkeep
    slot = b % 2
    o_ref[0] = jnp.where(keep != 0.0, xbuf_ref[slot], 0.0)


def kernel(x, x_len, freq_starts, freq_lengths, time_starts, time_lengths):
    grid_spec = pltpu.PrefetchScalarGridSpec(
        num_scalar_prefetch=5,
        grid=(_B,),
        in_specs=[pl.BlockSpec(memory_space=pl.ANY)],
        out_specs=pl.BlockSpec((1, _F, _T), lambda b, *_: (b, 0, 0)),
        scratch_shapes=[
            pltpu.VMEM((2, _F, _T), jnp.float32),
            pltpu.VMEM((1, _T), jnp.float32),
            pltpu.SemaphoreType.DMA((2,)),
        ],
    )
    return pl.pallas_call(
        _body,
        grid_spec=grid_spec,
        out_shape=jax.ShapeDtypeStruct((_B, _F, _T), jnp.float32),
    )(x_len, freq_starts, freq_lengths, time_starts, time_lengths, x)


# speedup vs baseline: 1.3328x; 1.3328x over previous
import jax
import jax.numpy as jnp
from jax.experimental import pallas as pl
from jax.experimental.pallas import tpu as pltpu

_B, _F, _T = 128, 80, 4096
_FREQ_MASKS = 2
_TIME_MASKS = 10
_BB = 4


def _body(xlen_ref, fs_ref, fl_ref, ts_ref, tl_ref, x_ref, o_ref, tkeep_ref):
    g = pl.program_id(0)

    @pl.when(g == 0)
    def _():
        t_io = jax.lax.broadcasted_iota(jnp.int32, (1, _T), 1)
        tk = jnp.ones((1, _T), jnp.float32)
        for i in range(_TIME_MASKS):
            s = ts_ref[i]
            e = s + tl_ref[i]
            tk = jnp.where((t_io >= s) & (t_io < e), 0.0, tk)
        tkeep_ref[...] = tk

    f_io = jax.lax.broadcasted_iota(jnp.int32, (_F, 1), 0)
    fkeep = jnp.ones((_F, 1), jnp.float32)
    for i in range(_FREQ_MASKS):
        s = fs_ref[i]
        e = s + fl_ref[i]
        fkeep = jnp.where((f_io >= s) & (f_io < e), 0.0, fkeep)

    t_io = jax.lax.broadcasted_iota(jnp.int32, (1, _T), 1)
    for lb in range(_BB):
        xl = xlen_ref[g * _BB + lb]
        tkeep = jnp.where(t_io < xl, tkeep_ref[...], 1.0)
        o_ref[lb] = x_ref[lb] * (fkeep * tkeep)


def kernel(x, x_len, freq_starts, freq_lengths, time_starts, time_lengths):
    grid_spec = pltpu.PrefetchScalarGridSpec(
        num_scalar_prefetch=5,
        grid=(_B // _BB,),
        in_specs=[pl.BlockSpec((_BB, _F, _T), lambda g, *_: (g, 0, 0))],
        out_specs=pl.BlockSpec((_BB, _F, _T), lambda g, *_: (g, 0, 0)),
        scratch_shapes=[pltpu.VMEM((1, _T), jnp.float32)],
    )
    return pl.pallas_call(
        _body,
        grid_spec=grid_spec,
        out_shape=jax.ShapeDtypeStruct((_B, _F, _T), jnp.float32),
    )(x_len, freq_starts, freq_lengths, time_starts, time_lengths, x)


# speedup vs baseline: 1.5190x; 1.1397x over previous
import jax
import jax.numpy as jnp
from jax.experimental import pallas as pl
from jax.experimental.pallas import tpu as pltpu

_B, _F, _T = 128, 80, 4096
_FREQ_MASKS = 2
_TIME_MASKS = 10
_BB = 4
_S = 8
_NCHUNK = _F // _S
_NG = _B // _BB


def _spans(fs_ref, fl_ref):
    s0, l0 = fs_ref[0], fl_ref[0]
    s1, l1 = fs_ref[1], fl_ref[1]
    e0, e1 = s0 + l0, s1 + l1
    first = s0 <= s1
    a_s = jnp.where(first, s0, s1)
    a_e = jnp.where(first, e0, e1)
    b_s = jnp.where(first, s1, s0)
    b_e = jnp.where(first, e1, e0)
    merged = b_s <= a_e
    i0s = a_s
    i0e = jnp.where(merged, jnp.maximum(a_e, b_e), a_e)
    i1s = jnp.where(merged, _F, b_s)
    i1e = jnp.where(merged, _F, b_e)
    return ((jnp.int32(0), i0s), (i0e, i1s), (i1e, jnp.int32(_F)))


def _issue(spans, g_src, slot, x_ref, xbuf_ref, sem_ref, do_wait):
    for ss, se in spans:
        base = (ss // _S) * _S
        for k in range(_NCHUNK):
            start = base + k * _S

            @pl.when(start < se)
            def _(start=start):
                start = pl.multiple_of(start, _S)
                cp = pltpu.make_async_copy(
                    x_ref.at[pl.ds(g_src * _BB, _BB), pl.ds(start, _S), :],
                    xbuf_ref.at[slot, :, pl.ds(start, _S), :],
                    sem_ref.at[slot],
                )
                if do_wait:
                    cp.wait()
                else:
                    cp.start()


def _body(xlen_ref, fs_ref, fl_ref, ts_ref, tl_ref, x_ref, o_ref,
          xbuf_ref, tkeep_ref, sem_ref):
    g = pl.program_id(0)
    spans = _spans(fs_ref, fl_ref)

    @pl.when(g == 0)
    def _():
        _issue(spans, 0, 0, x_ref, xbuf_ref, sem_ref, do_wait=False)
        t_io = jax.lax.broadcasted_iota(jnp.int32, (1, _T), 1)
        tk = jnp.ones((1, _T), jnp.float32)
        for i in range(_TIME_MASKS):
            s = ts_ref[i]
            e = s + tl_ref[i]
            tk = jnp.where((t_io >= s) & (t_io < e), 0.0, tk)
        tkeep_ref[...] = tk

    @pl.when(g + 1 < _NG)
    def _():
        _issue(spans, g + 1, (g + 1) % 2, x_ref, xbuf_ref, sem_ref,
               do_wait=False)

    _issue(spans, g, g % 2, x_ref, xbuf_ref, sem_ref, do_wait=True)

    f_io = jax.lax.broadcasted_iota(jnp.int32, (_F, 1), 0)
    fkeep = jnp.ones((_F, 1), jnp.float32)
    for i in range(_FREQ_MASKS):
        s = fs_ref[i]
        e = s + fl_ref[i]
        fkeep = jnp.where((f_io >= s) & (f_io < e), 0.0, fkeep)

    t_io = jax.lax.broadcasted_iota(jnp.int32, (1, _T), 1)
    slot = g % 2
    for lb in range(_BB):
        xl = xlen_ref[g * _BB + lb]
        tkeep = jnp.where(t_io < xl, tkeep_ref[...], 1.0)
        keep = fkeep * tkeep
        o_ref[lb] = jnp.where(keep != 0.0, xbuf_ref[slot, lb], 0.0)


def kernel(x, x_len, freq_starts, freq_lengths, time_starts, time_lengths):
    grid_spec = pltpu.PrefetchScalarGridSpec(
        num_scalar_prefetch=5,
        grid=(_NG,),
        in_specs=[pl.BlockSpec(memory_space=pl.ANY)],
        out_specs=pl.BlockSpec((_BB, _F, _T), lambda g, *_: (g, 0, 0)),
        scratch_shapes=[
            pltpu.VMEM((2, _BB, _F, _T), jnp.float32),
            pltpu.VMEM((1, _T), jnp.float32),
            pltpu.SemaphoreType.DMA((2,)),
        ],
    )
    return pl.pallas_call(
        _body,
        grid_spec=grid_spec,
        out_shape=jax.ShapeDtypeStruct((_B, _F, _T), jnp.float32),
    )(x_len, freq_starts, freq_lengths, time_starts, time_lengths, x)
